# Initial kernel scaffold; baseline (speedup 1.0000x reference)
import jax
import jax.numpy as jnp
from jax.experimental import pallas as pl


def kernel(x, x_a, sadj, fadj, params):
    raise NotImplementedError("write your pallas kernel here")


# speedup vs baseline: 2.6104x; 2.6104x over previous
import functools
import math

import jax
import jax.numpy as jnp
from jax.experimental import pallas as pl

_BN_INV = 1.0 / math.sqrt(1.0 + 1e-5)


def _pick_block(n, want):
    if n % want == 0:
        return want
    for b in range(min(want, n), 0, -1):
        if n % b == 0 and (b % 8 == 0 or b == n):
            return b
    return n


def _proj_body(x_ref, xa_ref, wsc_ref, wfc_ref, wcl_ref, us_ref, uf_ref):
    xb = x_ref[...]
    xab = xa_ref[...]
    pcl = jnp.dot(xab, wcl_ref[...], preferred_element_type=jnp.float32)
    us_ref[...] = jnp.concatenate(
        [jnp.dot(xb, wsc_ref[...], preferred_element_type=jnp.float32), pcl], axis=1)
    uf_ref[...] = jnp.concatenate(
        [jnp.dot(xb, wfc_ref[...], preferred_element_type=jnp.float32), pcl], axis=1)


def _project(x, x_a, wsc, wfc, wcl):
    n, nf = x.shape
    k = wsc.shape[1] + wcl.shape[1]
    bm = _pick_block(n, 2000)
    grid = (n // bm,)
    return pl.pallas_call(
        _proj_body,
        grid=grid,
        in_specs=[
            pl.BlockSpec((bm, nf), lambda i: (i, 0)),
            pl.BlockSpec((bm, nf), lambda i: (i, 0)),
            pl.BlockSpec(wsc.shape, lambda i: (0, 0)),
            pl.BlockSpec(wfc.shape, lambda i: (0, 0)),
            pl.BlockSpec(wcl.shape, lambda i: (0, 0)),
        ],
        out_specs=[
            pl.BlockSpec((bm, k), lambda i: (i, 0)),
            pl.BlockSpec((bm, k), lambda i: (i, 0)),
        ],
        out_shape=[
            jax.ShapeDtypeStruct((n, k), jnp.float32),
            jax.ShapeDtypeStruct((n, k), jnp.float32),
        ],
    )(x, x_a, wsc, wfc, wcl)


def _passA_body(a_ref, u_ref, b_ref, bd_ref, v_ref):
    acc = jnp.dot(a_ref[...], u_ref[...], preferred_element_type=jnp.float32)
    h = jnp.maximum(acc + b_ref[...], 0.0)
    v_ref[...] = jnp.dot(h, bd_ref[...], preferred_element_type=jnp.float32)


def _passA(adj, u, b1cat, bd):
    n = adj.shape[0]
    ku = u.shape[1]
    kv = bd.shape[1]
    bm = _pick_block(n, 400)
    return pl.pallas_call(
        _passA_body,
        grid=(n // bm,),
        in_specs=[
            pl.BlockSpec((bm, n), lambda i: (i, 0)),
            pl.BlockSpec((n, ku), lambda i: (0, 0)),
            pl.BlockSpec((1, ku), lambda i: (0, 0)),
            pl.BlockSpec((ku, kv), lambda i: (0, 0)),
        ],
        out_specs=pl.BlockSpec((bm, kv), lambda i: (i, 0)),
        out_shape=jax.ShapeDtypeStruct((n, kv), jnp.float32),
    )(adj, u, b1cat, bd)


def _passB_body(a_ref, v_ref, b_ref, z_ref):
    z_ref[...] = (jnp.dot(a_ref[...], v_ref[...], preferred_element_type=jnp.float32)
                  + b_ref[...])


def _passB(adj, v, b2cat):
    n = adj.shape[0]
    kv = v.shape[1]
    bm = _pick_block(n, 400)
    return pl.pallas_call(
        _passB_body,
        grid=(n // bm,),
        in_specs=[
            pl.BlockSpec((bm, n), lambda i: (i, 0)),
            pl.BlockSpec((n, kv), lambda i: (0, 0)),
            pl.BlockSpec((1, kv), lambda i: (0, 0)),
        ],
        out_specs=pl.BlockSpec((bm, kv), lambda i: (i, 0)),
        out_shape=jax.ShapeDtypeStruct((n, kv), jnp.float32),
    )(adj, v, b2cat)


def _passC_body(a_ref, e_ref, g_ref, *, half):
    a = a_ref[...]
    vsum = jnp.dot(a, e_ref[...], preferred_element_type=jnp.float32)
    rs = jnp.sum(a, axis=1, keepdims=True)
    ge = vsum / rs
    ge1 = ge[:, :half]
    ge2 = ge[:, half:]
    n1 = jnp.maximum(jnp.sqrt(jnp.sum(ge1 * ge1, axis=1, keepdims=True)), 1e-12)
    n2 = jnp.maximum(jnp.sqrt(jnp.sum(ge2 * ge2, axis=1, keepdims=True)), 1e-12)
    g_ref[...] = jax.nn.sigmoid(jnp.concatenate([ge1 / n1, ge2 / n2], axis=1))


def _passC(adj, e):
    n = adj.shape[0]
    ke = e.shape[1]
    bm = _pick_block(n, 400)
    return pl.pallas_call(
        functools.partial(_passC_body, half=ke // 2),
        grid=(n // bm,),
        in_specs=[
            pl.BlockSpec((bm, n), lambda i: (i, 0)),
            pl.BlockSpec((n, ke), lambda i: (0, 0)),
        ],
        out_specs=pl.BlockSpec((bm, ke), lambda i: (i, 0)),
        out_shape=jax.ShapeDtypeStruct((n, ke), jnp.float32),
    )(adj, e)


def _head_body(zs_ref, zf_ref, gs_ref, gf_ref,
               attw1_ref, attb1_ref, attw2_ref,
               mlpw_ref, mlpb_ref,
               decw1_ref, decb1_ref, g1_ref, be1_ref,
               wpi_ref, bpi_ref, wd_ref, bd_ref, wm_ref, bm_ref,
               discw_ref, discb_ref,
               adw1_ref, adb1_ref, adw2_ref,
               emb_ref, pi_ref, disp_ref, mean_ref, rets_ref, retf_ref):
    zs = zs_ref[...]
    zf = zf_ref[...]
    emb_s, com_s, emb_sa = zs[:, 0:32], zs[:, 32:64], zs[:, 64:96]
    emb_f, com_f, emb_fa = zf[:, 0:32], zf[:, 32:64], zf[:, 64:96]
    com = (com_s + com_f) * 0.5

    attw1 = attw1_ref[...]
    attb1 = attb1_ref[...]
    attw2 = attw2_ref[...]

    def att_score(z):
        t = jnp.tanh(jnp.dot(z, attw1, preferred_element_type=jnp.float32) + attb1)
        return jnp.sum(t * attw2, axis=1, keepdims=True)

    w0, w1, w2 = att_score(emb_s), att_score(emb_f), att_score(com)
    m = jnp.maximum(jnp.maximum(w0, w1), w2)
    e0, e1, e2 = jnp.exp(w0 - m), jnp.exp(w1 - m), jnp.exp(w2 - m)
    emb = (e0 * emb_s + e1 * emb_f + e2 * com) / (e0 + e1 + e2)
    emb = jnp.dot(emb, mlpw_ref[...], preferred_element_type=jnp.float32) + mlpb_ref[...]
    emb_ref[...] = emb

    hz = jnp.dot(emb, decw1_ref[...], preferred_element_type=jnp.float32) + decb1_ref[...]
    emb_z = jnp.maximum(hz * _BN_INV * g1_ref[...] + be1_ref[...], 0.0)
    pi_ref[...] = jax.nn.sigmoid(
        jnp.dot(emb_z, wpi_ref[...], preferred_element_type=jnp.float32) + bpi_ref[...])
    disp_ref[...] = jnp.clip(jax.nn.softplus(
        jnp.dot(emb_z, wd_ref[...], preferred_element_type=jnp.float32) + bd_ref[...]),
        1e-4, 1e4)
    mean_ref[...] = jnp.clip(jnp.exp(
        jnp.dot(emb_z, wm_ref[...], preferred_element_type=jnp.float32) + bm_ref[...]),
        1e-5, 1e6)

    gs = gs_ref[...]
    gf = gf_ref[...]
    g_s, g_sa = gs[:, :32], gs[:, 32:]
    g_f, g_fa = gf[:, :32], gf[:, 32:]
    w = discw_ref[...]
    b = discb_ref[...]
    cs_w = jnp.dot(com_s, w, preferred_element_type=jnp.float32)
    sa_w = jnp.dot(emb_sa, w, preferred_element_type=jnp.float32)
    cf_w = jnp.dot(com_f, w, preferred_element_type=jnp.float32)
    fa_w = jnp.dot(emb_fa, w, preferred_element_type=jnp.float32)

    def sc(t, c):
        return jnp.sum(t * c, axis=1, keepdims=True) + b

    rs1 = jnp.concatenate([sc(cs_w, g_s), sc(sa_w, g_s)], axis=1)
    rs2 = jnp.concatenate([sc(sa_w, g_sa), sc(cs_w, g_sa)], axis=1)
    rf1 = jnp.concatenate([sc(cf_w, g_f), sc(fa_w, g_f)], axis=1)
    rf2 = jnp.concatenate([sc(fa_w, g_fa), sc(cf_w, g_fa)], axis=1)

    adw1 = adw1_ref[...]
    adb1 = adb1_ref[...]
    adw2 = adw2_ref[...]

    def ad_score(z2):
        t = z2[:, 0:1] * adw1[0:1, :] + z2[:, 1:2] * adw1[1:2, :] + adb1
        return jnp.sum(jnp.tanh(t) * adw2, axis=1, keepdims=True)

    def att_d(r1, r2):
        w1_ = ad_score(r1)
        w2_ = ad_score(r2)
        mm = jnp.maximum(w1_, w2_)
        ee1 = jnp.exp(w1_ - mm)
        ee2 = jnp.exp(w2_ - mm)
        return (ee1 * r1 + ee2 * r2) / (ee1 + ee2)

    rets_ref[...] = att_d(rs1, rs2)
    retf_ref[...] = att_d(rf1, rf2)


def _head(zs, zf, gs, gf, p):
    n = zs.shape[0]
    bm = _pick_block(n, 2000)
    r2 = lambda a: a.reshape(1, -1)
    att, attd, mlp, dec, zinb, disc = (p['att'], p['att_d'], p['MLP'], p['dec'],
                                       p['zinb'], p['disc'])
    weights = [
        att['W1'], r2(att['b1']), r2(att['W2']),
        mlp['W'], r2(mlp['b']),
        dec['W1'], r2(dec['b1']), r2(dec['g1']), r2(dec['be1']),
        zinb['Wpi'], r2(zinb['bpi']), zinb['Wd'], r2(zinb['bd']),
        zinb['Wm'], r2(zinb['bm']),
        disc['W'][0], r2(disc['b']),
        attd['W1'], r2(attd['b1']), r2(attd['W2']),
    ]
    in_specs = ([pl.BlockSpec((bm, zs.shape[1]), lambda i: (i, 0)),
                 pl.BlockSpec((bm, zf.shape[1]), lambda i: (i, 0)),
                 pl.BlockSpec((bm, gs.shape[1]), lambda i: (i, 0)),
                 pl.BlockSpec((bm, gf.shape[1]), lambda i: (i, 0))]
                + [pl.BlockSpec(wt.shape, lambda i: (0, 0)) for wt in weights])
    out_specs = [
        pl.BlockSpec((bm, 32), lambda i: (i, 0)),
        pl.BlockSpec((bm, 128), lambda i: (i, 0)),
        pl.BlockSpec((bm, 128), lambda i: (i, 0)),
        pl.BlockSpec((bm, 128), lambda i: (i, 0)),
        pl.BlockSpec((bm, 2), lambda i: (i, 0)),
        pl.BlockSpec((bm, 2), lambda i: (i, 0)),
    ]
    out_shape = [
        jax.ShapeDtypeStruct((n, 32), jnp.float32),
        jax.ShapeDtypeStruct((n, 128), jnp.float32),
        jax.ShapeDtypeStruct((n, 128), jnp.float32),
        jax.ShapeDtypeStruct((n, 128), jnp.float32),
        jax.ShapeDtypeStruct((n, 2), jnp.float32),
        jax.ShapeDtypeStruct((n, 2), jnp.float32),
    ]
    return pl.pallas_call(
        _head_body,
        grid=(n // bm,),
        in_specs=in_specs,
        out_specs=out_specs,
        out_shape=out_shape,
    )(zs, zf, gs, gf, *weights)


def kernel(x, x_a, sadj, fadj, params):
    p = params
    f32 = jnp.float32

    wsc = jnp.concatenate([p['SGCN']['W1'], p['CGCN']['W1']], axis=1)
    wfc = jnp.concatenate([p['FGCN']['W1'], p['CGCN']['W1']], axis=1)
    u_s, u_f = _project(x, x_a, wsc, wfc, p['CLGCN']['W1'])

    def bdiag(ws):
        z = jnp.zeros((64, 32), f32)
        return jnp.concatenate([
            jnp.concatenate([ws[0], z, z], axis=1),
            jnp.concatenate([z, ws[1], z], axis=1),
            jnp.concatenate([z, z, ws[2]], axis=1)], axis=0)

    b1_s = jnp.concatenate([p['SGCN']['b1'], p['CGCN']['b1'], p['CLGCN']['b1']])
    b1_f = jnp.concatenate([p['FGCN']['b1'], p['CGCN']['b1'], p['CLGCN']['b1']])
    b2_s = jnp.concatenate([p['SGCN']['b2'], p['CGCN']['b2'], p['CLGCN']['b2']])
    b2_f = jnp.concatenate([p['FGCN']['b2'], p['CGCN']['b2'], p['CLGCN']['b2']])
    bd_s = bdiag([p['SGCN']['W2'], p['CGCN']['W2'], p['CLGCN']['W2']])
    bd_f = bdiag([p['FGCN']['W2'], p['CGCN']['W2'], p['CLGCN']['W2']])

    v_s = _passA(sadj, u_s, b1_s.reshape(1, -1), bd_s)
    v_f = _passA(fadj, u_f, b1_f.reshape(1, -1), bd_f)
    z_s = _passB(sadj, v_s, b2_s.reshape(1, -1))
    z_f = _passB(fadj, v_f, b2_f.reshape(1, -1))

    g_s = _passC(sadj, z_s[:, 32:96])
    g_f = _passC(fadj, z_f[:, 32:96])

    emb, pi, disp, mean, ret_s, ret_f = _head(z_s, z_f, g_s, g_f, p)
    com_s = z_s[:, 32:64]
    com_f = z_f[:, 32:64]
    return (com_s, com_f, emb, pi, disp, mean, ret_s, ret_f)


# speedup vs baseline: 2.6883x; 1.0298x over previous
import functools
import math

import jax
import jax.numpy as jnp
from jax.experimental import pallas as pl

_BN_INV = 1.0 / math.sqrt(1.0 + 1e-5)


def _pick_block(n, want):
    if n % want == 0:
        return want
    for b in range(min(want, n), 0, -1):
        if n % b == 0 and (b % 8 == 0 or b == n):
            return b
    return n


def _proj_body(x_ref, xa_ref, wsc_ref, wfc_ref, wcl_ref, us_ref, uf_ref):
    xb = x_ref[...]
    xab = xa_ref[...]
    pcl = jnp.dot(xab, wcl_ref[...], preferred_element_type=jnp.float32)
    us_ref[...] = jnp.concatenate(
        [jnp.dot(xb, wsc_ref[...], preferred_element_type=jnp.float32), pcl], axis=1)
    uf_ref[...] = jnp.concatenate(
        [jnp.dot(xb, wfc_ref[...], preferred_element_type=jnp.float32), pcl], axis=1)


def _project(x, x_a, wsc, wfc, wcl):
    n, nf = x.shape
    k = wsc.shape[1] + wcl.shape[1]
    bm = _pick_block(n, 2000)
    grid = (n // bm,)
    return pl.pallas_call(
        _proj_body,
        grid=grid,
        in_specs=[
            pl.BlockSpec((bm, nf), lambda i: (i, 0)),
            pl.BlockSpec((bm, nf), lambda i: (i, 0)),
            pl.BlockSpec(wsc.shape, lambda i: (0, 0)),
            pl.BlockSpec(wfc.shape, lambda i: (0, 0)),
            pl.BlockSpec(wcl.shape, lambda i: (0, 0)),
        ],
        out_specs=[
            pl.BlockSpec((bm, k), lambda i: (i, 0)),
            pl.BlockSpec((bm, k), lambda i: (i, 0)),
        ],
        out_shape=[
            jax.ShapeDtypeStruct((n, k), jnp.float32),
            jax.ShapeDtypeStruct((n, k), jnp.float32),
        ],
    )(x, x_a, wsc, wfc, wcl)


def _passA_body(a_ref, u_ref, b_ref, bd_ref, v_ref, abf_ref):
    abf = a_ref[...].astype(jnp.bfloat16)
    abf_ref[...] = abf
    acc = jnp.dot(abf, u_ref[...], preferred_element_type=jnp.float32)
    h = jnp.maximum(acc + b_ref[...], 0.0)
    v_ref[...] = jnp.dot(h, bd_ref[...], preferred_element_type=jnp.float32)


def _passA(adj, u, b1cat, bd):
    n = adj.shape[0]
    ku = u.shape[1]
    kv = bd.shape[1]
    bm = _pick_block(n, 400)
    return pl.pallas_call(
        _passA_body,
        grid=(n // bm,),
        in_specs=[
            pl.BlockSpec((bm, n), lambda i: (i, 0)),
            pl.BlockSpec((n, ku), lambda i: (0, 0)),
            pl.BlockSpec((1, ku), lambda i: (0, 0)),
            pl.BlockSpec((ku, kv), lambda i: (0, 0)),
        ],
        out_specs=[pl.BlockSpec((bm, kv), lambda i: (i, 0)),
                   pl.BlockSpec((bm, n), lambda i: (i, 0))],
        out_shape=[jax.ShapeDtypeStruct((n, kv), jnp.float32),
                   jax.ShapeDtypeStruct((n, n), jnp.bfloat16)],
    )(adj, u.astype(jnp.bfloat16), b1cat, bd)


def _passB_body(a_ref, v_ref, b_ref, z_ref):
    z_ref[...] = (jnp.dot(a_ref[...], v_ref[...], preferred_element_type=jnp.float32)
                  + b_ref[...])


def _passB(adj, v, b2cat):
    n = adj.shape[0]
    kv = v.shape[1]
    bm = _pick_block(n, 400)
    return pl.pallas_call(
        _passB_body,
        grid=(n // bm,),
        in_specs=[
            pl.BlockSpec((bm, n), lambda i: (i, 0)),
            pl.BlockSpec((n, kv), lambda i: (0, 0)),
            pl.BlockSpec((1, kv), lambda i: (0, 0)),
        ],
        out_specs=pl.BlockSpec((bm, kv), lambda i: (i, 0)),
        out_shape=jax.ShapeDtypeStruct((n, kv), jnp.float32),
    )(adj, v.astype(jnp.bfloat16), b2cat)


def _passC_body(a_ref, e_ref, g_ref, *, half):
    a = a_ref[...]
    vsum = jnp.dot(a, e_ref[...], preferred_element_type=jnp.float32)
    rs = jnp.sum(a.astype(jnp.float32), axis=1, keepdims=True)
    ge = vsum / rs
    ge1 = ge[:, :half]
    ge2 = ge[:, half:]
    n1 = jnp.maximum(jnp.sqrt(jnp.sum(ge1 * ge1, axis=1, keepdims=True)), 1e-12)
    n2 = jnp.maximum(jnp.sqrt(jnp.sum(ge2 * ge2, axis=1, keepdims=True)), 1e-12)
    g_ref[...] = jax.nn.sigmoid(jnp.concatenate([ge1 / n1, ge2 / n2], axis=1))


def _passC(adj, e):
    n = adj.shape[0]
    ke = e.shape[1]
    bm = _pick_block(n, 400)
    return pl.pallas_call(
        functools.partial(_passC_body, half=ke // 2),
        grid=(n // bm,),
        in_specs=[
            pl.BlockSpec((bm, n), lambda i: (i, 0)),
            pl.BlockSpec((n, ke), lambda i: (0, 0)),
        ],
        out_specs=pl.BlockSpec((bm, ke), lambda i: (i, 0)),
        out_shape=jax.ShapeDtypeStruct((n, ke), jnp.float32),
    )(adj, e.astype(jnp.bfloat16))


def _head_body(zs_ref, zf_ref, gs_ref, gf_ref,
               attw1_ref, attb1_ref, attw2_ref,
               mlpw_ref, mlpb_ref,
               decw1_ref, decb1_ref, g1_ref, be1_ref,
               wpi_ref, bpi_ref, wd_ref, bd_ref, wm_ref, bm_ref,
               discw_ref, discb_ref,
               adw1_ref, adb1_ref, adw2_ref,
               emb_ref, pi_ref, disp_ref, mean_ref, rets_ref, retf_ref):
    zs = zs_ref[...]
    zf = zf_ref[...]
    emb_s, com_s, emb_sa = zs[:, 0:32], zs[:, 32:64], zs[:, 64:96]
    emb_f, com_f, emb_fa = zf[:, 0:32], zf[:, 32:64], zf[:, 64:96]
    com = (com_s + com_f) * 0.5

    attw1 = attw1_ref[...]
    attb1 = attb1_ref[...]
    attw2 = attw2_ref[...]

    def att_score(z):
        t = jnp.tanh(jnp.dot(z, attw1, preferred_element_type=jnp.float32) + attb1)
        return jnp.sum(t * attw2, axis=1, keepdims=True)

    w0, w1, w2 = att_score(emb_s), att_score(emb_f), att_score(com)
    m = jnp.maximum(jnp.maximum(w0, w1), w2)
    e0, e1, e2 = jnp.exp(w0 - m), jnp.exp(w1 - m), jnp.exp(w2 - m)
    emb = (e0 * emb_s + e1 * emb_f + e2 * com) / (e0 + e1 + e2)
    emb = jnp.dot(emb, mlpw_ref[...], preferred_element_type=jnp.float32) + mlpb_ref[...]
    emb_ref[...] = emb

    hz = jnp.dot(emb, decw1_ref[...], preferred_element_type=jnp.float32) + decb1_ref[...]
    emb_z = jnp.maximum(hz * _BN_INV * g1_ref[...] + be1_ref[...], 0.0)
    pi_ref[...] = jax.nn.sigmoid(
        jnp.dot(emb_z, wpi_ref[...], preferred_element_type=jnp.float32) + bpi_ref[...])
    disp_ref[...] = jnp.clip(jax.nn.softplus(
        jnp.dot(emb_z, wd_ref[...], preferred_element_type=jnp.float32) + bd_ref[...]),
        1e-4, 1e4)
    mean_ref[...] = jnp.clip(jnp.exp(
        jnp.dot(emb_z, wm_ref[...], preferred_element_type=jnp.float32) + bm_ref[...]),
        1e-5, 1e6)

    gs = gs_ref[...]
    gf = gf_ref[...]
    g_s, g_sa = gs[:, :32], gs[:, 32:]
    g_f, g_fa = gf[:, :32], gf[:, 32:]
    w = discw_ref[...]
    b = discb_ref[...]
    cs_w = jnp.dot(com_s, w, preferred_element_type=jnp.float32)
    sa_w = jnp.dot(emb_sa, w, preferred_element_type=jnp.float32)
    cf_w = jnp.dot(com_f, w, preferred_element_type=jnp.float32)
    fa_w = jnp.dot(emb_fa, w, preferred_element_type=jnp.float32)

    def sc(t, c):
        return jnp.sum(t * c, axis=1, keepdims=True) + b

    rs1 = jnp.concatenate([sc(cs_w, g_s), sc(sa_w, g_s)], axis=1)
    rs2 = jnp.concatenate([sc(sa_w, g_sa), sc(cs_w, g_sa)], axis=1)
    rf1 = jnp.concatenate([sc(cf_w, g_f), sc(fa_w, g_f)], axis=1)
    rf2 = jnp.concatenate([sc(fa_w, g_fa), sc(cf_w, g_fa)], axis=1)

    adw1 = adw1_ref[...]
    adb1 = adb1_ref[...]
    adw2 = adw2_ref[...]

    def ad_score(z2):
        t = z2[:, 0:1] * adw1[0:1, :] + z2[:, 1:2] * adw1[1:2, :] + adb1
        return jnp.sum(jnp.tanh(t) * adw2, axis=1, keepdims=True)

    def att_d(r1, r2):
        w1_ = ad_score(r1)
        w2_ = ad_score(r2)
        mm = jnp.maximum(w1_, w2_)
        ee1 = jnp.exp(w1_ - mm)
        ee2 = jnp.exp(w2_ - mm)
        return (ee1 * r1 + ee2 * r2) / (ee1 + ee2)

    rets_ref[...] = att_d(rs1, rs2)
    retf_ref[...] = att_d(rf1, rf2)


def _head(zs, zf, gs, gf, p):
    n = zs.shape[0]
    bm = _pick_block(n, 2000)
    r2 = lambda a: a.reshape(1, -1)
    att, attd, mlp, dec, zinb, disc = (p['att'], p['att_d'], p['MLP'], p['dec'],
                                       p['zinb'], p['disc'])
    weights = [
        att['W1'], r2(att['b1']), r2(att['W2']),
        mlp['W'], r2(mlp['b']),
        dec['W1'], r2(dec['b1']), r2(dec['g1']), r2(dec['be1']),
        zinb['Wpi'], r2(zinb['bpi']), zinb['Wd'], r2(zinb['bd']),
        zinb['Wm'], r2(zinb['bm']),
        disc['W'][0], r2(disc['b']),
        attd['W1'], r2(attd['b1']), r2(attd['W2']),
    ]
    in_specs = ([pl.BlockSpec((bm, zs.shape[1]), lambda i: (i, 0)),
                 pl.BlockSpec((bm, zf.shape[1]), lambda i: (i, 0)),
                 pl.BlockSpec((bm, gs.shape[1]), lambda i: (i, 0)),
                 pl.BlockSpec((bm, gf.shape[1]), lambda i: (i, 0))]
                + [pl.BlockSpec(wt.shape, lambda i: (0, 0)) for wt in weights])
    out_specs = [
        pl.BlockSpec((bm, 32), lambda i: (i, 0)),
        pl.BlockSpec((bm, 128), lambda i: (i, 0)),
        pl.BlockSpec((bm, 128), lambda i: (i, 0)),
        pl.BlockSpec((bm, 128), lambda i: (i, 0)),
        pl.BlockSpec((bm, 2), lambda i: (i, 0)),
        pl.BlockSpec((bm, 2), lambda i: (i, 0)),
    ]
    out_shape = [
        jax.ShapeDtypeStruct((n, 32), jnp.float32),
        jax.ShapeDtypeStruct((n, 128), jnp.float32),
        jax.ShapeDtypeStruct((n, 128), jnp.float32),
        jax.ShapeDtypeStruct((n, 128), jnp.float32),
        jax.ShapeDtypeStruct((n, 2), jnp.float32),
        jax.ShapeDtypeStruct((n, 2), jnp.float32),
    ]
    return pl.pallas_call(
        _head_body,
        grid=(n // bm,),
        in_specs=in_specs,
        out_specs=out_specs,
        out_shape=out_shape,
    )(zs, zf, gs, gf, *weights)


def kernel(x, x_a, sadj, fadj, params):
    p = params
    f32 = jnp.float32

    wsc = jnp.concatenate([p['SGCN']['W1'], p['CGCN']['W1']], axis=1)
    wfc = jnp.concatenate([p['FGCN']['W1'], p['CGCN']['W1']], axis=1)
    u_s, u_f = _project(x, x_a, wsc, wfc, p['CLGCN']['W1'])

    def bdiag(ws):
        z = jnp.zeros((64, 32), f32)
        return jnp.concatenate([
            jnp.concatenate([ws[0], z, z], axis=1),
            jnp.concatenate([z, ws[1], z], axis=1),
            jnp.concatenate([z, z, ws[2]], axis=1)], axis=0)

    b1_s = jnp.concatenate([p['SGCN']['b1'], p['CGCN']['b1'], p['CLGCN']['b1']])
    b1_f = jnp.concatenate([p['FGCN']['b1'], p['CGCN']['b1'], p['CLGCN']['b1']])
    b2_s = jnp.concatenate([p['SGCN']['b2'], p['CGCN']['b2'], p['CLGCN']['b2']])
    b2_f = jnp.concatenate([p['FGCN']['b2'], p['CGCN']['b2'], p['CLGCN']['b2']])
    bd_s = bdiag([p['SGCN']['W2'], p['CGCN']['W2'], p['CLGCN']['W2']])
    bd_f = bdiag([p['FGCN']['W2'], p['CGCN']['W2'], p['CLGCN']['W2']])

    v_s, sadj_bf = _passA(sadj, u_s, b1_s.reshape(1, -1), bd_s)
    v_f, fadj_bf = _passA(fadj, u_f, b1_f.reshape(1, -1), bd_f)
    z_s = _passB(sadj_bf, v_s, b2_s.reshape(1, -1))
    z_f = _passB(fadj_bf, v_f, b2_f.reshape(1, -1))

    g_s = _passC(sadj_bf, z_s[:, 32:96])
    g_f = _passC(fadj_bf, z_f[:, 32:96])

    emb, pi, disp, mean, ret_s, ret_f = _head(z_s, z_f, g_s, g_f, p)
    com_s = z_s[:, 32:64]
    com_f = z_f[:, 32:64]
    return (com_s, com_f, emb, pi, disp, mean, ret_s, ret_f)


# speedup vs baseline: 2.6960x; 1.0029x over previous
import functools
import math

import jax
import jax.numpy as jnp
from jax.experimental import pallas as pl

_BN_INV = 1.0 / math.sqrt(1.0 + 1e-5)


def _pick_block(n, want):
    if n % want == 0:
        return want
    for b in range(min(want, n), 0, -1):
        if n % b == 0 and (b % 8 == 0 or b == n):
            return b
    return n


def _proj_body(x_ref, xa_ref, wsc_ref, wfc_ref, wcl_ref, us_ref, uf_ref):
    xb = x_ref[...]
    xab = xa_ref[...]
    pcl = jnp.dot(xab, wcl_ref[...], preferred_element_type=jnp.float32)
    us_ref[...] = jnp.concatenate(
        [jnp.dot(xb, wsc_ref[...], preferred_element_type=jnp.float32), pcl], axis=1)
    uf_ref[...] = jnp.concatenate(
        [jnp.dot(xb, wfc_ref[...], preferred_element_type=jnp.float32), pcl], axis=1)


def _project(x, x_a, wsc, wfc, wcl):
    n, nf = x.shape
    k = wsc.shape[1] + wcl.shape[1]
    bm = _pick_block(n, 2000)
    grid = (n // bm,)
    return pl.pallas_call(
        _proj_body,
        grid=grid,
        in_specs=[
            pl.BlockSpec((bm, nf), lambda i: (i, 0)),
            pl.BlockSpec((bm, nf), lambda i: (i, 0)),
            pl.BlockSpec(wsc.shape, lambda i: (0, 0)),
            pl.BlockSpec(wfc.shape, lambda i: (0, 0)),
            pl.BlockSpec(wcl.shape, lambda i: (0, 0)),
        ],
        out_specs=[
            pl.BlockSpec((bm, k), lambda i: (i, 0)),
            pl.BlockSpec((bm, k), lambda i: (i, 0)),
        ],
        out_shape=[
            jax.ShapeDtypeStruct((n, k), jnp.float32),
            jax.ShapeDtypeStruct((n, k), jnp.float32),
        ],
    )(x, x_a, wsc, wfc, wcl)


def _passA_body(a_ref, u_ref, b_ref, bd_ref, v_ref, abf_ref, rs_ref):
    a = a_ref[...]
    abf = a.astype(jnp.bfloat16)
    abf_ref[...] = abf
    rs_ref[...] = jnp.sum(a, axis=1, keepdims=True)
    acc = jnp.dot(abf, u_ref[...], preferred_element_type=jnp.float32)
    h = jnp.maximum(acc + b_ref[...], 0.0)
    v_ref[...] = jnp.dot(h, bd_ref[...], preferred_element_type=jnp.float32)


def _passA(adj, u, b1cat, bd):
    n = adj.shape[0]
    ku = u.shape[1]
    kv = bd.shape[1]
    bm = _pick_block(n, 400)
    return pl.pallas_call(
        _passA_body,
        grid=(n // bm,),
        in_specs=[
            pl.BlockSpec((bm, n), lambda i: (i, 0)),
            pl.BlockSpec((n, ku), lambda i: (0, 0)),
            pl.BlockSpec((1, ku), lambda i: (0, 0)),
            pl.BlockSpec((ku, kv), lambda i: (0, 0)),
        ],
        out_specs=[pl.BlockSpec((bm, kv), lambda i: (i, 0)),
                   pl.BlockSpec((bm, n), lambda i: (i, 0)),
                   pl.BlockSpec((bm, 1), lambda i: (i, 0))],
        out_shape=[jax.ShapeDtypeStruct((n, kv), jnp.float32),
                   jax.ShapeDtypeStruct((n, n), jnp.bfloat16),
                   jax.ShapeDtypeStruct((n, 1), jnp.float32)],
    )(adj, u.astype(jnp.bfloat16), b1cat, bd)


def _passB_body(a_ref, v_ref, b_ref, z_ref):
    z_ref[...] = (jnp.dot(a_ref[...], v_ref[...], preferred_element_type=jnp.float32)
                  + b_ref[...])


def _passB(adj, v, b2cat):
    n = adj.shape[0]
    kv = v.shape[1]
    bm = _pick_block(n, 400)
    return pl.pallas_call(
        _passB_body,
        grid=(n // bm,),
        in_specs=[
            pl.BlockSpec((bm, n), lambda i: (i, 0)),
            pl.BlockSpec((n, kv), lambda i: (0, 0)),
            pl.BlockSpec((1, kv), lambda i: (0, 0)),
        ],
        out_specs=pl.BlockSpec((bm, kv), lambda i: (i, 0)),
        out_shape=jax.ShapeDtypeStruct((n, kv), jnp.float32),
    )(adj, v.astype(jnp.bfloat16), b2cat)


def _passC_body(a_ref, e_ref, rs_ref, g_ref, *, half):
    a = a_ref[...]
    vsum = jnp.dot(a, e_ref[...], preferred_element_type=jnp.float32)
    ge = vsum / rs_ref[...]
    ge1 = ge[:, :half]
    ge2 = ge[:, half:]
    n1 = jnp.maximum(jnp.sqrt(jnp.sum(ge1 * ge1, axis=1, keepdims=True)), 1e-12)
    n2 = jnp.maximum(jnp.sqrt(jnp.sum(ge2 * ge2, axis=1, keepdims=True)), 1e-12)
    g_ref[...] = jax.nn.sigmoid(jnp.concatenate([ge1 / n1, ge2 / n2], axis=1))


def _passC(adj, e, rs):
    n = adj.shape[0]
    ke = e.shape[1]
    bm = _pick_block(n, 400)
    return pl.pallas_call(
        functools.partial(_passC_body, half=ke // 2),
        grid=(n // bm,),
        in_specs=[
            pl.BlockSpec((bm, n), lambda i: (i, 0)),
            pl.BlockSpec((n, ke), lambda i: (0, 0)),
            pl.BlockSpec((bm, 1), lambda i: (i, 0)),
        ],
        out_specs=pl.BlockSpec((bm, ke), lambda i: (i, 0)),
        out_shape=jax.ShapeDtypeStruct((n, ke), jnp.float32),
    )(adj, e.astype(jnp.bfloat16), rs)


def _head_body(zs_ref, zf_ref, gs_ref, gf_ref,
               attw1_ref, attb1_ref, attw2_ref,
               mlpw_ref, mlpb_ref,
               decw1_ref, decb1_ref, g1_ref, be1_ref,
               wpi_ref, bpi_ref, wd_ref, bd_ref, wm_ref, bm_ref,
               discw_ref, discb_ref,
               adw1_ref, adb1_ref, adw2_ref,
               emb_ref, pi_ref, disp_ref, mean_ref, rets_ref, retf_ref):
    zs = zs_ref[...]
    zf = zf_ref[...]
    emb_s, com_s, emb_sa = zs[:, 0:32], zs[:, 32:64], zs[:, 64:96]
    emb_f, com_f, emb_fa = zf[:, 0:32], zf[:, 32:64], zf[:, 64:96]
    com = (com_s + com_f) * 0.5

    attw1 = attw1_ref[...]
    attb1 = attb1_ref[...]
    attw2 = attw2_ref[...]

    def att_score(z):
        t = jnp.tanh(jnp.dot(z, attw1, preferred_element_type=jnp.float32) + attb1)
        return jnp.sum(t * attw2, axis=1, keepdims=True)

    w0, w1, w2 = att_score(emb_s), att_score(emb_f), att_score(com)
    m = jnp.maximum(jnp.maximum(w0, w1), w2)
    e0, e1, e2 = jnp.exp(w0 - m), jnp.exp(w1 - m), jnp.exp(w2 - m)
    emb = (e0 * emb_s + e1 * emb_f + e2 * com) / (e0 + e1 + e2)
    emb = jnp.dot(emb, mlpw_ref[...], preferred_element_type=jnp.float32) + mlpb_ref[...]
    emb_ref[...] = emb

    hz = jnp.dot(emb, decw1_ref[...], preferred_element_type=jnp.float32) + decb1_ref[...]
    emb_z = jnp.maximum(hz * _BN_INV * g1_ref[...] + be1_ref[...], 0.0)
    pi_ref[...] = jax.nn.sigmoid(
        jnp.dot(emb_z, wpi_ref[...], preferred_element_type=jnp.float32) + bpi_ref[...])
    disp_ref[...] = jnp.clip(jax.nn.softplus(
        jnp.dot(emb_z, wd_ref[...], preferred_element_type=jnp.float32) + bd_ref[...]),
        1e-4, 1e4)
    mean_ref[...] = jnp.clip(jnp.exp(
        jnp.dot(emb_z, wm_ref[...], preferred_element_type=jnp.float32) + bm_ref[...]),
        1e-5, 1e6)

    gs = gs_ref[...]
    gf = gf_ref[...]
    g_s, g_sa = gs[:, :32], gs[:, 32:]
    g_f, g_fa = gf[:, :32], gf[:, 32:]
    w = discw_ref[...]
    b = discb_ref[...]
    cs_w = jnp.dot(com_s, w, preferred_element_type=jnp.float32)
    sa_w = jnp.dot(emb_sa, w, preferred_element_type=jnp.float32)
    cf_w = jnp.dot(com_f, w, preferred_element_type=jnp.float32)
    fa_w = jnp.dot(emb_fa, w, preferred_element_type=jnp.float32)

    def sc(t, c):
        return jnp.sum(t * c, axis=1, keepdims=True) + b

    rs1 = jnp.concatenate([sc(cs_w, g_s), sc(sa_w, g_s)], axis=1)
    rs2 = jnp.concatenate([sc(sa_w, g_sa), sc(cs_w, g_sa)], axis=1)
    rf1 = jnp.concatenate([sc(cf_w, g_f), sc(fa_w, g_f)], axis=1)
    rf2 = jnp.concatenate([sc(fa_w, g_fa), sc(cf_w, g_fa)], axis=1)

    adw1 = adw1_ref[...]
    adb1 = adb1_ref[...]
    adw2 = adw2_ref[...]

    def ad_score(z2):
        t = z2[:, 0:1] * adw1[0:1, :] + z2[:, 1:2] * adw1[1:2, :] + adb1
        return jnp.sum(jnp.tanh(t) * adw2, axis=1, keepdims=True)

    def att_d(r1, r2):
        w1_ = ad_score(r1)
        w2_ = ad_score(r2)
        mm = jnp.maximum(w1_, w2_)
        ee1 = jnp.exp(w1_ - mm)
        ee2 = jnp.exp(w2_ - mm)
        return (ee1 * r1 + ee2 * r2) / (ee1 + ee2)

    rets_ref[...] = att_d(rs1, rs2)
    retf_ref[...] = att_d(rf1, rf2)


def _head(zs, zf, gs, gf, p):
    n = zs.shape[0]
    bm = _pick_block(n, 2000)
    r2 = lambda a: a.reshape(1, -1)
    att, attd, mlp, dec, zinb, disc = (p['att'], p['att_d'], p['MLP'], p['dec'],
                                       p['zinb'], p['disc'])
    weights = [
        att['W1'], r2(att['b1']), r2(att['W2']),
        mlp['W'], r2(mlp['b']),
        dec['W1'], r2(dec['b1']), r2(dec['g1']), r2(dec['be1']),
        zinb['Wpi'], r2(zinb['bpi']), zinb['Wd'], r2(zinb['bd']),
        zinb['Wm'], r2(zinb['bm']),
        disc['W'][0], r2(disc['b']),
        attd['W1'], r2(attd['b1']), r2(attd['W2']),
    ]
    in_specs = ([pl.BlockSpec((bm, zs.shape[1]), lambda i: (i, 0)),
                 pl.BlockSpec((bm, zf.shape[1]), lambda i: (i, 0)),
                 pl.BlockSpec((bm, gs.shape[1]), lambda i: (i, 0)),
                 pl.BlockSpec((bm, gf.shape[1]), lambda i: (i, 0))]
                + [pl.BlockSpec(wt.shape, lambda i: (0, 0)) for wt in weights])
    out_specs = [
        pl.BlockSpec((bm, 32), lambda i: (i, 0)),
        pl.BlockSpec((bm, 128), lambda i: (i, 0)),
        pl.BlockSpec((bm, 128), lambda i: (i, 0)),
        pl.BlockSpec((bm, 128), lambda i: (i, 0)),
        pl.BlockSpec((bm, 2), lambda i: (i, 0)),
        pl.BlockSpec((bm, 2), lambda i: (i, 0)),
    ]
    out_shape = [
        jax.ShapeDtypeStruct((n, 32), jnp.float32),
        jax.ShapeDtypeStruct((n, 128), jnp.float32),
        jax.ShapeDtypeStruct((n, 128), jnp.float32),
        jax.ShapeDtypeStruct((n, 128), jnp.float32),
        jax.ShapeDtypeStruct((n, 2), jnp.float32),
        jax.ShapeDtypeStruct((n, 2), jnp.float32),
    ]
    return pl.pallas_call(
        _head_body,
        grid=(n // bm,),
        in_specs=in_specs,
        out_specs=out_specs,
        out_shape=out_shape,
    )(zs, zf, gs, gf, *weights)


def kernel(x, x_a, sadj, fadj, params):
    p = params
    f32 = jnp.float32

    wsc = jnp.concatenate([p['SGCN']['W1'], p['CGCN']['W1']], axis=1)
    wfc = jnp.concatenate([p['FGCN']['W1'], p['CGCN']['W1']], axis=1)
    u_s, u_f = _project(x, x_a, wsc, wfc, p['CLGCN']['W1'])

    def bdiag(ws):
        z = jnp.zeros((64, 32), f32)
        return jnp.concatenate([
            jnp.concatenate([ws[0], z, z], axis=1),
            jnp.concatenate([z, ws[1], z], axis=1),
            jnp.concatenate([z, z, ws[2]], axis=1)], axis=0)

    b1_s = jnp.concatenate([p['SGCN']['b1'], p['CGCN']['b1'], p['CLGCN']['b1']])
    b1_f = jnp.concatenate([p['FGCN']['b1'], p['CGCN']['b1'], p['CLGCN']['b1']])
    b2_s = jnp.concatenate([p['SGCN']['b2'], p['CGCN']['b2'], p['CLGCN']['b2']])
    b2_f = jnp.concatenate([p['FGCN']['b2'], p['CGCN']['b2'], p['CLGCN']['b2']])
    bd_s = bdiag([p['SGCN']['W2'], p['CGCN']['W2'], p['CLGCN']['W2']])
    bd_f = bdiag([p['FGCN']['W2'], p['CGCN']['W2'], p['CLGCN']['W2']])

    v_s, sadj_bf, rs_s = _passA(sadj, u_s, b1_s.reshape(1, -1), bd_s)
    v_f, fadj_bf, rs_f = _passA(fadj, u_f, b1_f.reshape(1, -1), bd_f)
    z_s = _passB(sadj_bf, v_s, b2_s.reshape(1, -1))
    z_f = _passB(fadj_bf, v_f, b2_f.reshape(1, -1))

    g_s = _passC(sadj_bf, z_s[:, 32:96], rs_s)
    g_f = _passC(fadj_bf, z_f[:, 32:96], rs_f)

    emb, pi, disp, mean, ret_s, ret_f = _head(z_s, z_f, g_s, g_f, p)
    com_s = z_s[:, 32:64]
    com_f = z_f[:, 32:64]
    return (com_s, com_f, emb, pi, disp, mean, ret_s, ret_f)


# speedup vs baseline: 2.7028x; 1.0025x over previous
import functools
import math

import jax
import jax.numpy as jnp
from jax.experimental import pallas as pl

_BN_INV = 1.0 / math.sqrt(1.0 + 1e-5)


def _pick_block(n, want):
    if n % want == 0:
        return want
    for b in range(min(want, n), 0, -1):
        if n % b == 0 and (b % 8 == 0 or b == n):
            return b
    return n


def _proj_body(x_ref, xa_ref, wsc_ref, wfc_ref, wcl_ref, us_ref, uf_ref):
    xb = x_ref[...]
    xab = xa_ref[...]
    pcl = jnp.dot(xab, wcl_ref[...], preferred_element_type=jnp.float32)
    us_ref[...] = jnp.concatenate(
        [jnp.dot(xb, wsc_ref[...], preferred_element_type=jnp.float32), pcl], axis=1)
    uf_ref[...] = jnp.concatenate(
        [jnp.dot(xb, wfc_ref[...], preferred_element_type=jnp.float32), pcl], axis=1)


def _project(x, x_a, wsc, wfc, wcl):
    n, nf = x.shape
    k = wsc.shape[1] + wcl.shape[1]
    bm = _pick_block(n, 2000)
    grid = (n // bm,)
    return pl.pallas_call(
        _proj_body,
        grid=grid,
        in_specs=[
            pl.BlockSpec((bm, nf), lambda i: (i, 0)),
            pl.BlockSpec((bm, nf), lambda i: (i, 0)),
            pl.BlockSpec(wsc.shape, lambda i: (0, 0)),
            pl.BlockSpec(wfc.shape, lambda i: (0, 0)),
            pl.BlockSpec(wcl.shape, lambda i: (0, 0)),
        ],
        out_specs=[
            pl.BlockSpec((bm, k), lambda i: (i, 0)),
            pl.BlockSpec((bm, k), lambda i: (i, 0)),
        ],
        out_shape=[
            jax.ShapeDtypeStruct((n, k), jnp.float32),
            jax.ShapeDtypeStruct((n, k), jnp.float32),
        ],
    )(x, x_a, wsc, wfc, wcl)


def _passA_body(a_ref, u_ref, b_ref, bd_ref, v_ref, abf_ref, rs_ref):
    a = a_ref[...]
    abf = a.astype(jnp.bfloat16)
    abf_ref[...] = abf
    rs_ref[...] = jnp.sum(a, axis=1, keepdims=True)
    acc = jnp.dot(abf, u_ref[...], preferred_element_type=jnp.float32)
    h = jnp.maximum(acc + b_ref[...], 0.0)
    v_ref[...] = jnp.dot(h, bd_ref[...], preferred_element_type=jnp.float32)


def _passA(adj, u, b1cat, bd):
    n = adj.shape[0]
    ku = u.shape[1]
    kv = bd.shape[1]
    bm = _pick_block(n, 400)
    return pl.pallas_call(
        _passA_body,
        grid=(n // bm,),
        in_specs=[
            pl.BlockSpec((bm, n), lambda i: (i, 0)),
            pl.BlockSpec((n, ku), lambda i: (0, 0)),
            pl.BlockSpec((1, ku), lambda i: (0, 0)),
            pl.BlockSpec((ku, kv), lambda i: (0, 0)),
        ],
        out_specs=[pl.BlockSpec((bm, kv), lambda i: (i, 0)),
                   pl.BlockSpec((bm, n), lambda i: (i, 0)),
                   pl.BlockSpec((bm, 1), lambda i: (i, 0))],
        out_shape=[jax.ShapeDtypeStruct((n, kv), jnp.float32),
                   jax.ShapeDtypeStruct((n, n), jnp.bfloat16),
                   jax.ShapeDtypeStruct((n, 1), jnp.float32)],
    )(adj, u.astype(jnp.bfloat16), b1cat, bd)


def _passB_body(a_ref, v_ref, b_ref, z_ref):
    z_ref[...] = (jnp.dot(a_ref[...], v_ref[...], preferred_element_type=jnp.float32)
                  + b_ref[...])


def _passB(adj, v, b2cat):
    n = adj.shape[0]
    kv = v.shape[1]
    bm = _pick_block(n, 800)
    return pl.pallas_call(
        _passB_body,
        grid=(n // bm,),
        in_specs=[
            pl.BlockSpec((bm, n), lambda i: (i, 0)),
            pl.BlockSpec((n, kv), lambda i: (0, 0)),
            pl.BlockSpec((1, kv), lambda i: (0, 0)),
        ],
        out_specs=pl.BlockSpec((bm, kv), lambda i: (i, 0)),
        out_shape=jax.ShapeDtypeStruct((n, kv), jnp.float32),
    )(adj, v.astype(jnp.bfloat16), b2cat)


def _passC_body(a_ref, e_ref, rs_ref, g_ref, *, half):
    a = a_ref[...]
    vsum = jnp.dot(a, e_ref[...], preferred_element_type=jnp.float32)
    ge = vsum / rs_ref[...]
    ge1 = ge[:, :half]
    ge2 = ge[:, half:]
    n1 = jnp.maximum(jnp.sqrt(jnp.sum(ge1 * ge1, axis=1, keepdims=True)), 1e-12)
    n2 = jnp.maximum(jnp.sqrt(jnp.sum(ge2 * ge2, axis=1, keepdims=True)), 1e-12)
    g_ref[...] = jax.nn.sigmoid(jnp.concatenate([ge1 / n1, ge2 / n2], axis=1))


def _passC(adj, e, rs):
    n = adj.shape[0]
    ke = e.shape[1]
    bm = _pick_block(n, 800)
    return pl.pallas_call(
        functools.partial(_passC_body, half=ke // 2),
        grid=(n // bm,),
        in_specs=[
            pl.BlockSpec((bm, n), lambda i: (i, 0)),
            pl.BlockSpec((n, ke), lambda i: (0, 0)),
            pl.BlockSpec((bm, 1), lambda i: (i, 0)),
        ],
        out_specs=pl.BlockSpec((bm, ke), lambda i: (i, 0)),
        out_shape=jax.ShapeDtypeStruct((n, ke), jnp.float32),
    )(adj, e.astype(jnp.bfloat16), rs)


def _head_body(zs_ref, zf_ref, gs_ref, gf_ref,
               attw1_ref, attb1_ref, attw2_ref,
               mlpw_ref, mlpb_ref,
               decw1_ref, decb1_ref, g1_ref, be1_ref,
               wpi_ref, bpi_ref, wd_ref, bd_ref, wm_ref, bm_ref,
               discw_ref, discb_ref,
               adw1_ref, adb1_ref, adw2_ref,
               emb_ref, pi_ref, disp_ref, mean_ref, rets_ref, retf_ref):
    zs = zs_ref[...]
    zf = zf_ref[...]
    emb_s, com_s, emb_sa = zs[:, 0:32], zs[:, 32:64], zs[:, 64:96]
    emb_f, com_f, emb_fa = zf[:, 0:32], zf[:, 32:64], zf[:, 64:96]
    com = (com_s + com_f) * 0.5

    attw1 = attw1_ref[...]
    attb1 = attb1_ref[...]
    attw2 = attw2_ref[...]

    def att_score(z):
        t = jnp.tanh(jnp.dot(z, attw1, preferred_element_type=jnp.float32) + attb1)
        return jnp.sum(t * attw2, axis=1, keepdims=True)

    w0, w1, w2 = att_score(emb_s), att_score(emb_f), att_score(com)
    m = jnp.maximum(jnp.maximum(w0, w1), w2)
    e0, e1, e2 = jnp.exp(w0 - m), jnp.exp(w1 - m), jnp.exp(w2 - m)
    emb = (e0 * emb_s + e1 * emb_f + e2 * com) / (e0 + e1 + e2)
    emb = jnp.dot(emb, mlpw_ref[...], preferred_element_type=jnp.float32) + mlpb_ref[...]
    emb_ref[...] = emb

    hz = jnp.dot(emb, decw1_ref[...], preferred_element_type=jnp.float32) + decb1_ref[...]
    emb_z = jnp.maximum(hz * _BN_INV * g1_ref[...] + be1_ref[...], 0.0)
    pi_ref[...] = jax.nn.sigmoid(
        jnp.dot(emb_z, wpi_ref[...], preferred_element_type=jnp.float32) + bpi_ref[...])
    disp_ref[...] = jnp.clip(jax.nn.softplus(
        jnp.dot(emb_z, wd_ref[...], preferred_element_type=jnp.float32) + bd_ref[...]),
        1e-4, 1e4)
    mean_ref[...] = jnp.clip(jnp.exp(
        jnp.dot(emb_z, wm_ref[...], preferred_element_type=jnp.float32) + bm_ref[...]),
        1e-5, 1e6)

    gs = gs_ref[...]
    gf = gf_ref[...]
    g_s, g_sa = gs[:, :32], gs[:, 32:]
    g_f, g_fa = gf[:, :32], gf[:, 32:]
    w = discw_ref[...]
    b = discb_ref[...]
    cs_w = jnp.dot(com_s, w, preferred_element_type=jnp.float32)
    sa_w = jnp.dot(emb_sa, w, preferred_element_type=jnp.float32)
    cf_w = jnp.dot(com_f, w, preferred_element_type=jnp.float32)
    fa_w = jnp.dot(emb_fa, w, preferred_element_type=jnp.float32)

    def sc(t, c):
        return jnp.sum(t * c, axis=1, keepdims=True) + b

    rs1 = jnp.concatenate([sc(cs_w, g_s), sc(sa_w, g_s)], axis=1)
    rs2 = jnp.concatenate([sc(sa_w, g_sa), sc(cs_w, g_sa)], axis=1)
    rf1 = jnp.concatenate([sc(cf_w, g_f), sc(fa_w, g_f)], axis=1)
    rf2 = jnp.concatenate([sc(fa_w, g_fa), sc(cf_w, g_fa)], axis=1)

    adw1 = adw1_ref[...]
    adb1 = adb1_ref[...]
    adw2 = adw2_ref[...]

    def ad_score(z2):
        t = z2[:, 0:1] * adw1[0:1, :] + z2[:, 1:2] * adw1[1:2, :] + adb1
        return jnp.sum(jnp.tanh(t) * adw2, axis=1, keepdims=True)

    def att_d(r1, r2):
        w1_ = ad_score(r1)
        w2_ = ad_score(r2)
        mm = jnp.maximum(w1_, w2_)
        ee1 = jnp.exp(w1_ - mm)
        ee2 = jnp.exp(w2_ - mm)
        return (ee1 * r1 + ee2 * r2) / (ee1 + ee2)

    rets_ref[...] = att_d(rs1, rs2)
    retf_ref[...] = att_d(rf1, rf2)


def _head(zs, zf, gs, gf, p):
    n = zs.shape[0]
    bm = _pick_block(n, 2000)
    r2 = lambda a: a.reshape(1, -1)
    att, attd, mlp, dec, zinb, disc = (p['att'], p['att_d'], p['MLP'], p['dec'],
                                       p['zinb'], p['disc'])
    weights = [
        att['W1'], r2(att['b1']), r2(att['W2']),
        mlp['W'], r2(mlp['b']),
        dec['W1'], r2(dec['b1']), r2(dec['g1']), r2(dec['be1']),
        zinb['Wpi'], r2(zinb['bpi']), zinb['Wd'], r2(zinb['bd']),
        zinb['Wm'], r2(zinb['bm']),
        disc['W'][0], r2(disc['b']),
        attd['W1'], r2(attd['b1']), r2(attd['W2']),
    ]
    in_specs = ([pl.BlockSpec((bm, zs.shape[1]), lambda i: (i, 0)),
                 pl.BlockSpec((bm, zf.shape[1]), lambda i: (i, 0)),
                 pl.BlockSpec((bm, gs.shape[1]), lambda i: (i, 0)),
                 pl.BlockSpec((bm, gf.shape[1]), lambda i: (i, 0))]
                + [pl.BlockSpec(wt.shape, lambda i: (0, 0)) for wt in weights])
    out_specs = [
        pl.BlockSpec((bm, 32), lambda i: (i, 0)),
        pl.BlockSpec((bm, 128), lambda i: (i, 0)),
        pl.BlockSpec((bm, 128), lambda i: (i, 0)),
        pl.BlockSpec((bm, 128), lambda i: (i, 0)),
        pl.BlockSpec((bm, 2), lambda i: (i, 0)),
        pl.BlockSpec((bm, 2), lambda i: (i, 0)),
    ]
    out_shape = [
        jax.ShapeDtypeStruct((n, 32), jnp.float32),
        jax.ShapeDtypeStruct((n, 128), jnp.float32),
        jax.ShapeDtypeStruct((n, 128), jnp.float32),
        jax.ShapeDtypeStruct((n, 128), jnp.float32),
        jax.ShapeDtypeStruct((n, 2), jnp.float32),
        jax.ShapeDtypeStruct((n, 2), jnp.float32),
    ]
    return pl.pallas_call(
        _head_body,
        grid=(n // bm,),
        in_specs=in_specs,
        out_specs=out_specs,
        out_shape=out_shape,
    )(zs, zf, gs, gf, *weights)


def kernel(x, x_a, sadj, fadj, params):
    p = params
    f32 = jnp.float32

    wsc = jnp.concatenate([p['SGCN']['W1'], p['CGCN']['W1']], axis=1)
    wfc = jnp.concatenate([p['FGCN']['W1'], p['CGCN']['W1']], axis=1)
    u_s, u_f = _project(x, x_a, wsc, wfc, p['CLGCN']['W1'])

    def bdiag(ws):
        z = jnp.zeros((64, 32), f32)
        return jnp.concatenate([
            jnp.concatenate([ws[0], z, z], axis=1),
            jnp.concatenate([z, ws[1], z], axis=1),
            jnp.concatenate([z, z, ws[2]], axis=1)], axis=0)

    b1_s = jnp.concatenate([p['SGCN']['b1'], p['CGCN']['b1'], p['CLGCN']['b1']])
    b1_f = jnp.concatenate([p['FGCN']['b1'], p['CGCN']['b1'], p['CLGCN']['b1']])
    b2_s = jnp.concatenate([p['SGCN']['b2'], p['CGCN']['b2'], p['CLGCN']['b2']])
    b2_f = jnp.concatenate([p['FGCN']['b2'], p['CGCN']['b2'], p['CLGCN']['b2']])
    bd_s = bdiag([p['SGCN']['W2'], p['CGCN']['W2'], p['CLGCN']['W2']])
    bd_f = bdiag([p['FGCN']['W2'], p['CGCN']['W2'], p['CLGCN']['W2']])

    v_s, sadj_bf, rs_s = _passA(sadj, u_s, b1_s.reshape(1, -1), bd_s)
    v_f, fadj_bf, rs_f = _passA(fadj, u_f, b1_f.reshape(1, -1), bd_f)
    z_s = _passB(sadj_bf, v_s, b2_s.reshape(1, -1))
    z_f = _passB(fadj_bf, v_f, b2_f.reshape(1, -1))

    g_s = _passC(sadj_bf, z_s[:, 32:96], rs_s)
    g_f = _passC(fadj_bf, z_f[:, 32:96], rs_f)

    emb, pi, disp, mean, ret_s, ret_f = _head(z_s, z_f, g_s, g_f, p)
    com_s = z_s[:, 32:64]
    com_f = z_f[:, 32:64]
    return (com_s, com_f, emb, pi, disp, mean, ret_s, ret_f)


# speedup vs baseline: 2.8511x; 1.0549x over previous
import functools
import math

import jax
import jax.numpy as jnp
from jax.experimental import pallas as pl

_BN_INV = 1.0 / math.sqrt(1.0 + 1e-5)


def _pick_block(n, want):
    if n % want == 0:
        return want
    for b in range(min(want, n), 0, -1):
        if n % b == 0 and (b % 8 == 0 or b == n):
            return b
    return n


def _proj_body(x_ref, xa_ref, wsc_ref, wfc_ref, wcl_ref, us_ref, uf_ref):
    xb = x_ref[...]
    xab = xa_ref[...]
    pcl = jnp.dot(xab, wcl_ref[...], preferred_element_type=jnp.float32)
    us_ref[...] = jnp.concatenate(
        [jnp.dot(xb, wsc_ref[...], preferred_element_type=jnp.float32), pcl], axis=1)
    uf_ref[...] = jnp.concatenate(
        [jnp.dot(xb, wfc_ref[...], preferred_element_type=jnp.float32), pcl], axis=1)


def _project(x, x_a, wsc, wfc, wcl):
    n, nf = x.shape
    k = wsc.shape[1] + wcl.shape[1]
    bm = _pick_block(n, 2000)
    grid = (n // bm,)
    return pl.pallas_call(
        _proj_body,
        grid=grid,
        in_specs=[
            pl.BlockSpec((bm, nf), lambda i: (i, 0)),
            pl.BlockSpec((bm, nf), lambda i: (i, 0)),
            pl.BlockSpec(wsc.shape, lambda i: (0, 0)),
            pl.BlockSpec(wfc.shape, lambda i: (0, 0)),
            pl.BlockSpec(wcl.shape, lambda i: (0, 0)),
        ],
        out_specs=[
            pl.BlockSpec((bm, k), lambda i: (i, 0)),
            pl.BlockSpec((bm, k), lambda i: (i, 0)),
        ],
        out_shape=[
            jax.ShapeDtypeStruct((n, k), jnp.float32),
            jax.ShapeDtypeStruct((n, k), jnp.float32),
        ],
    )(x, x_a, wsc, wfc, wcl)


def _passA_body(a_ref, u_ref, b_ref, bd_ref, v_ref, a8_ref, sa_ref, rs_ref):
    a = a_ref[...]
    rs_ref[...] = jnp.sum(a, axis=1, keepdims=True)
    rmax = jnp.maximum(jnp.max(a, axis=1, keepdims=True), 1e-30)
    sa_ref[...] = rmax * (1.0 / 127.0)
    a8_ref[...] = jnp.round(a * (127.0 / rmax)).astype(jnp.int8)
    acc = jnp.dot(a.astype(jnp.bfloat16), u_ref[...],
                  preferred_element_type=jnp.float32)
    h = jnp.maximum(acc + b_ref[...], 0.0)
    v_ref[...] = jnp.dot(h, bd_ref[...], preferred_element_type=jnp.float32)


def _passA(adj, u, b1cat, bd):
    n = adj.shape[0]
    ku = u.shape[1]
    kv = bd.shape[1]
    bm = _pick_block(n, 400)
    return pl.pallas_call(
        _passA_body,
        grid=(n // bm,),
        in_specs=[
            pl.BlockSpec((bm, n), lambda i: (i, 0)),
            pl.BlockSpec((n, ku), lambda i: (0, 0)),
            pl.BlockSpec((1, ku), lambda i: (0, 0)),
            pl.BlockSpec((ku, kv), lambda i: (0, 0)),
        ],
        out_specs=[pl.BlockSpec((bm, kv), lambda i: (i, 0)),
                   pl.BlockSpec((bm, n), lambda i: (i, 0)),
                   pl.BlockSpec((bm, 1), lambda i: (i, 0)),
                   pl.BlockSpec((bm, 1), lambda i: (i, 0))],
        out_shape=[jax.ShapeDtypeStruct((n, kv), jnp.float32),
                   jax.ShapeDtypeStruct((n, n), jnp.int8),
                   jax.ShapeDtypeStruct((n, 1), jnp.float32),
                   jax.ShapeDtypeStruct((n, 1), jnp.float32)],
    )(adj, u.astype(jnp.bfloat16), b1cat, bd)


def _passB_body(a_ref, v_ref, sa_ref, sv_ref, b_ref, z_ref):
    acc = jnp.dot(a_ref[...], v_ref[...], preferred_element_type=jnp.int32)
    z_ref[...] = acc.astype(jnp.float32) * sa_ref[...] * sv_ref[...] + b_ref[...]


def _quant_cols(v):
    sv = jnp.maximum(jnp.max(jnp.abs(v), axis=0, keepdims=True), 1e-30) / 127.0
    return jnp.round(v / sv).astype(jnp.int8), sv


def _passB(adj, v, sa, b2cat):
    n = adj.shape[0]
    kv = v.shape[1]
    bm = _pick_block(n, 800)
    v8, sv = _quant_cols(v)
    return pl.pallas_call(
        _passB_body,
        grid=(n // bm,),
        in_specs=[
            pl.BlockSpec((bm, n), lambda i: (i, 0)),
            pl.BlockSpec((n, kv), lambda i: (0, 0)),
            pl.BlockSpec((bm, 1), lambda i: (i, 0)),
            pl.BlockSpec((1, kv), lambda i: (0, 0)),
            pl.BlockSpec((1, kv), lambda i: (0, 0)),
        ],
        out_specs=pl.BlockSpec((bm, kv), lambda i: (i, 0)),
        out_shape=jax.ShapeDtypeStruct((n, kv), jnp.float32),
    )(adj, v8, sa, sv, b2cat)


def _passC_body(a_ref, e_ref, sa_ref, se_ref, rs_ref, g_ref, *, half):
    acc = jnp.dot(a_ref[...], e_ref[...], preferred_element_type=jnp.int32)
    vsum = acc.astype(jnp.float32) * sa_ref[...] * se_ref[...]
    ge = vsum / rs_ref[...]
    ge1 = ge[:, :half]
    ge2 = ge[:, half:]
    n1 = jnp.maximum(jnp.sqrt(jnp.sum(ge1 * ge1, axis=1, keepdims=True)), 1e-12)
    n2 = jnp.maximum(jnp.sqrt(jnp.sum(ge2 * ge2, axis=1, keepdims=True)), 1e-12)
    g_ref[...] = jax.nn.sigmoid(jnp.concatenate([ge1 / n1, ge2 / n2], axis=1))


def _passC(adj, e, sa, rs):
    n = adj.shape[0]
    ke = e.shape[1]
    bm = _pick_block(n, 800)
    e8, se = _quant_cols(e)
    return pl.pallas_call(
        functools.partial(_passC_body, half=ke // 2),
        grid=(n // bm,),
        in_specs=[
            pl.BlockSpec((bm, n), lambda i: (i, 0)),
            pl.BlockSpec((n, ke), lambda i: (0, 0)),
            pl.BlockSpec((bm, 1), lambda i: (i, 0)),
            pl.BlockSpec((1, ke), lambda i: (0, 0)),
            pl.BlockSpec((bm, 1), lambda i: (i, 0)),
        ],
        out_specs=pl.BlockSpec((bm, ke), lambda i: (i, 0)),
        out_shape=jax.ShapeDtypeStruct((n, ke), jnp.float32),
    )(adj, e8, sa, se, rs)


def _head_body(zs_ref, zf_ref, gs_ref, gf_ref,
               attw1_ref, attb1_ref, attw2_ref,
               mlpw_ref, mlpb_ref,
               decw1_ref, decb1_ref, g1_ref, be1_ref,
               wpi_ref, bpi_ref, wd_ref, bd_ref, wm_ref, bm_ref,
               discw_ref, discb_ref,
               adw1_ref, adb1_ref, adw2_ref,
               emb_ref, pi_ref, disp_ref, mean_ref, rets_ref, retf_ref):
    zs = zs_ref[...]
    zf = zf_ref[...]
    emb_s, com_s, emb_sa = zs[:, 0:32], zs[:, 32:64], zs[:, 64:96]
    emb_f, com_f, emb_fa = zf[:, 0:32], zf[:, 32:64], zf[:, 64:96]
    com = (com_s + com_f) * 0.5

    attw1 = attw1_ref[...]
    attb1 = attb1_ref[...]
    attw2 = attw2_ref[...]

    def att_score(z):
        t = jnp.tanh(jnp.dot(z, attw1, preferred_element_type=jnp.float32) + attb1)
        return jnp.sum(t * attw2, axis=1, keepdims=True)

    w0, w1, w2 = att_score(emb_s), att_score(emb_f), att_score(com)
    m = jnp.maximum(jnp.maximum(w0, w1), w2)
    e0, e1, e2 = jnp.exp(w0 - m), jnp.exp(w1 - m), jnp.exp(w2 - m)
    emb = (e0 * emb_s + e1 * emb_f + e2 * com) / (e0 + e1 + e2)
    emb = jnp.dot(emb, mlpw_ref[...], preferred_element_type=jnp.float32) + mlpb_ref[...]
    emb_ref[...] = emb

    hz = jnp.dot(emb, decw1_ref[...], preferred_element_type=jnp.float32) + decb1_ref[...]
    emb_z = jnp.maximum(hz * _BN_INV * g1_ref[...] + be1_ref[...], 0.0)
    pi_ref[...] = jax.nn.sigmoid(
        jnp.dot(emb_z, wpi_ref[...], preferred_element_type=jnp.float32) + bpi_ref[...])
    disp_ref[...] = jnp.clip(jax.nn.softplus(
        jnp.dot(emb_z, wd_ref[...], preferred_element_type=jnp.float32) + bd_ref[...]),
        1e-4, 1e4)
    mean_ref[...] = jnp.clip(jnp.exp(
        jnp.dot(emb_z, wm_ref[...], preferred_element_type=jnp.float32) + bm_ref[...]),
        1e-5, 1e6)

    gs = gs_ref[...]
    gf = gf_ref[...]
    g_s, g_sa = gs[:, :32], gs[:, 32:]
    g_f, g_fa = gf[:, :32], gf[:, 32:]
    w = discw_ref[...]
    b = discb_ref[...]
    cs_w = jnp.dot(com_s, w, preferred_element_type=jnp.float32)
    sa_w = jnp.dot(emb_sa, w, preferred_element_type=jnp.float32)
    cf_w = jnp.dot(com_f, w, preferred_element_type=jnp.float32)
    fa_w = jnp.dot(emb_fa, w, preferred_element_type=jnp.float32)

    def sc(t, c):
        return jnp.sum(t * c, axis=1, keepdims=True) + b

    rs1 = jnp.concatenate([sc(cs_w, g_s), sc(sa_w, g_s)], axis=1)
    rs2 = jnp.concatenate([sc(sa_w, g_sa), sc(cs_w, g_sa)], axis=1)
    rf1 = jnp.concatenate([sc(cf_w, g_f), sc(fa_w, g_f)], axis=1)
    rf2 = jnp.concatenate([sc(fa_w, g_fa), sc(cf_w, g_fa)], axis=1)

    adw1 = adw1_ref[...]
    adb1 = adb1_ref[...]
    adw2 = adw2_ref[...]

    def ad_score(z2):
        t = z2[:, 0:1] * adw1[0:1, :] + z2[:, 1:2] * adw1[1:2, :] + adb1
        return jnp.sum(jnp.tanh(t) * adw2, axis=1, keepdims=True)

    def att_d(r1, r2):
        w1_ = ad_score(r1)
        w2_ = ad_score(r2)
        mm = jnp.maximum(w1_, w2_)
        ee1 = jnp.exp(w1_ - mm)
        ee2 = jnp.exp(w2_ - mm)
        return (ee1 * r1 + ee2 * r2) / (ee1 + ee2)

    rets_ref[...] = att_d(rs1, rs2)
    retf_ref[...] = att_d(rf1, rf2)


def _head(zs, zf, gs, gf, p):
    n = zs.shape[0]
    bm = _pick_block(n, 2000)
    r2 = lambda a: a.reshape(1, -1)
    att, attd, mlp, dec, zinb, disc = (p['att'], p['att_d'], p['MLP'], p['dec'],
                                       p['zinb'], p['disc'])
    weights = [
        att['W1'], r2(att['b1']), r2(att['W2']),
        mlp['W'], r2(mlp['b']),
        dec['W1'], r2(dec['b1']), r2(dec['g1']), r2(dec['be1']),
        zinb['Wpi'], r2(zinb['bpi']), zinb['Wd'], r2(zinb['bd']),
        zinb['Wm'], r2(zinb['bm']),
        disc['W'][0], r2(disc['b']),
        attd['W1'], r2(attd['b1']), r2(attd['W2']),
    ]
    in_specs = ([pl.BlockSpec((bm, zs.shape[1]), lambda i: (i, 0)),
                 pl.BlockSpec((bm, zf.shape[1]), lambda i: (i, 0)),
                 pl.BlockSpec((bm, gs.shape[1]), lambda i: (i, 0)),
                 pl.BlockSpec((bm, gf.shape[1]), lambda i: (i, 0))]
                + [pl.BlockSpec(wt.shape, lambda i: (0, 0)) for wt in weights])
    out_specs = [
        pl.BlockSpec((bm, 32), lambda i: (i, 0)),
        pl.BlockSpec((bm, 128), lambda i: (i, 0)),
        pl.BlockSpec((bm, 128), lambda i: (i, 0)),
        pl.BlockSpec((bm, 128), lambda i: (i, 0)),
        pl.BlockSpec((bm, 2), lambda i: (i, 0)),
        pl.BlockSpec((bm, 2), lambda i: (i, 0)),
    ]
    out_shape = [
        jax.ShapeDtypeStruct((n, 32), jnp.float32),
        jax.ShapeDtypeStruct((n, 128), jnp.float32),
        jax.ShapeDtypeStruct((n, 128), jnp.float32),
        jax.ShapeDtypeStruct((n, 128), jnp.float32),
        jax.ShapeDtypeStruct((n, 2), jnp.float32),
        jax.ShapeDtypeStruct((n, 2), jnp.float32),
    ]
    return pl.pallas_call(
        _head_body,
        grid=(n // bm,),
        in_specs=in_specs,
        out_specs=out_specs,
        out_shape=out_shape,
    )(zs, zf, gs, gf, *weights)


def kernel(x, x_a, sadj, fadj, params):
    p = params
    f32 = jnp.float32

    wsc = jnp.concatenate([p['SGCN']['W1'], p['CGCN']['W1']], axis=1)
    wfc = jnp.concatenate([p['FGCN']['W1'], p['CGCN']['W1']], axis=1)
    u_s, u_f = _project(x, x_a, wsc, wfc, p['CLGCN']['W1'])

    def bdiag(ws):
        z = jnp.zeros((64, 32), f32)
        return jnp.concatenate([
            jnp.concatenate([ws[0], z, z], axis=1),
            jnp.concatenate([z, ws[1], z], axis=1),
            jnp.concatenate([z, z, ws[2]], axis=1)], axis=0)

    b1_s = jnp.concatenate([p['SGCN']['b1'], p['CGCN']['b1'], p['CLGCN']['b1']])
    b1_f = jnp.concatenate([p['FGCN']['b1'], p['CGCN']['b1'], p['CLGCN']['b1']])
    b2_s = jnp.concatenate([p['SGCN']['b2'], p['CGCN']['b2'], p['CLGCN']['b2']])
    b2_f = jnp.concatenate([p['FGCN']['b2'], p['CGCN']['b2'], p['CLGCN']['b2']])
    bd_s = bdiag([p['SGCN']['W2'], p['CGCN']['W2'], p['CLGCN']['W2']])
    bd_f = bdiag([p['FGCN']['W2'], p['CGCN']['W2'], p['CLGCN']['W2']])

    v_s, sadj8, sa_s, rs_s = _passA(sadj, u_s, b1_s.reshape(1, -1), bd_s)
    v_f, fadj8, sa_f, rs_f = _passA(fadj, u_f, b1_f.reshape(1, -1), bd_f)
    z_s = _passB(sadj8, v_s, sa_s, b2_s.reshape(1, -1))
    z_f = _passB(fadj8, v_f, sa_f, b2_f.reshape(1, -1))

    g_s = _passC(sadj8, z_s[:, 32:96], sa_s, rs_s)
    g_f = _passC(fadj8, z_f[:, 32:96], sa_f, rs_f)

    emb, pi, disp, mean, ret_s, ret_f = _head(z_s, z_f, g_s, g_f, p)
    com_s = z_s[:, 32:64]
    com_f = z_f[:, 32:64]
    return (com_s, com_f, emb, pi, disp, mean, ret_s, ret_f)
